# Initial kernel scaffold; baseline (speedup 1.0000x reference)
import jax
import jax.numpy as jnp
from jax.experimental import pallas as pl


def kernel(x, table):
    raise NotImplementedError("write your pallas kernel here")


# speedup vs baseline: 5.2638x; 5.2638x over previous
import functools

import jax
import jax.numpy as jnp
from jax import lax
from jax.experimental import pallas as pl
from jax.experimental.pallas import tpu as pltpu
from jax.experimental.pallas import tpu_sc as plsc

BUCKET_NUMBER = 1000
MIN_VALUE = 0.0
MAX_VALUE = 1.0
STEP = (MAX_VALUE - MIN_VALUE) / BUCKET_NUMBER
EMBED_DIM = 64

LANES = 16
IDX_BLK = 128
CHUNK = 512


def _make_kernel(B, D):
    info = plsc.get_sparse_core_info()
    NC, NS = info.num_cores, info.num_subcores
    NW = NC * NS
    assert B % (NW * CHUNK) == 0
    per_w = B // NW
    n_chunks = per_w // CHUNK
    n_blk = CHUNK // IDX_BLK

    mesh = plsc.VectorSubcoreMesh(core_axis_name="c", subcore_axis_name="s")

    @functools.partial(
        pl.kernel,
        out_type=jax.ShapeDtypeStruct((B, D), jnp.float32),
        mesh=mesh,
        scratch_types=[
            pltpu.VMEM((CHUNK,), jnp.float32),
            pltpu.VMEM((n_blk, IDX_BLK), jnp.int32),
            pltpu.VMEM((CHUNK, D), jnp.float32),
            pltpu.SemaphoreType.DMA,
        ],
        compiler_params=pltpu.CompilerParams(use_tc_tiling_on_sc=False),
    )
    def k(x_hbm, table_hbm, out_hbm, x_v, idx_v, rows_v, sem):
        wid = lax.axis_index("s") * NC + lax.axis_index("c")
        base = wid * per_w

        def chunk_body(g, carry):
            cbase = base + g * CHUNK
            pltpu.sync_copy(x_hbm.at[pl.ds(cbase, CHUNK)], x_v)
            for i in range(CHUNK // LANES):
                v = x_v[pl.ds(i * LANES, LANES)]
                t = (v - MIN_VALUE) / STEP
                idx = t.astype(jnp.int32)
                idx = jnp.minimum(jnp.maximum(idx, 0), BUCKET_NUMBER - 1)
                j, o = divmod(i * LANES, IDX_BLK)
                idx_v[j, pl.ds(o, LANES)] = idx
            copies = [
                pltpu.async_copy(
                    table_hbm.at[idx_v.at[j]],
                    rows_v.at[pl.ds(j * IDX_BLK, IDX_BLK)],
                    sem,
                )
                for j in range(n_blk)
            ]
            for c in copies:
                c.wait()
            pltpu.sync_copy(rows_v, out_hbm.at[pl.ds(cbase, CHUNK)])
            return carry

        lax.fori_loop(0, n_chunks, chunk_body, 0)

    return k


def kernel(x, table):
    if x.ndim == 2 and x.shape[1] == 1:
        x = jnp.squeeze(x, axis=-1)
    shape = x.shape
    B = x.size
    xf = x.reshape(B)
    out = _make_kernel(B, table.shape[1])(xf, table)
    return out.reshape(*shape, table.shape[1])
